# Initial kernel scaffold; baseline (speedup 1.0000x reference)
import jax
import jax.numpy as jnp
from jax.experimental import pallas as pl


def kernel(region_id, x, y, arrival_time, departure_time, region_table, Ws, bs, w0, b0, Wt, Bt):
    raise NotImplementedError("write your pallas kernel here")


# speedup vs baseline: 2.2663x; 2.2663x over previous
import functools

import numpy as np
import jax
import jax.numpy as jnp
from jax import lax
from jax.experimental import pallas as pl
from jax.experimental.pallas import tpu as pltpu
from jax.experimental.pallas import tpu_sc as plsc

_LAMBDA_MIN = 0.01
_LAMBDA_MAX = 10.0

_NUM_CORES = 2
_NUM_SUBCORES = 16
_NUM_WORKERS = _NUM_CORES * _NUM_SUBCORES


def _space2vec_consts(feat_dim: int, d_embed: int):
    num_scales = feat_dim // 6
    scales = _LAMBDA_MIN * (_LAMBDA_MAX / _LAMBDA_MIN) ** (
        jnp.arange(num_scales, dtype=jnp.float32) / max(num_scales - 1, 1))
    angles = jnp.array([0.0, 2.0 * np.pi / 3.0, 4.0 * np.pi / 3.0],
                       dtype=jnp.float32)
    dirx = jnp.cos(angles)
    diry = jnp.sin(angles)
    lanes = np.arange(feat_dim)
    dir_idx = np.asarray(lanes % 6 % 3)
    scale_idx = np.asarray(lanes // 6)
    pad = d_embed - feat_dim
    dx = jnp.concatenate([dirx[dir_idx], jnp.zeros((pad,), jnp.float32)])
    dy = jnp.concatenate([diry[dir_idx], jnp.zeros((pad,), jnp.float32)])
    scl = jnp.concatenate([scales[scale_idx], jnp.ones((pad,), jnp.float32)])
    sel = jnp.asarray(
        np.concatenate([(lanes % 6 < 3).astype(np.float32),
                        np.ones((pad,), np.float32)]))
    return dx, dy, scl, sel


def _sc_gather(table, idx_flat, n_tokens, d_embed):
    chunk = 128
    per_worker = n_tokens // _NUM_WORKERS
    n_chunks = per_worker // chunk
    idx3 = idx_flat.reshape(_NUM_WORKERS, n_chunks, chunk)
    mesh = plsc.VectorSubcoreMesh(core_axis_name="c", subcore_axis_name="s")

    @functools.partial(
        pl.kernel,
        out_type=jax.ShapeDtypeStruct((n_tokens, d_embed), jnp.float32),
        mesh=mesh,
        scratch_types=[
            pltpu.VMEM((n_chunks, chunk), jnp.int32),
            pltpu.VMEM((chunk, d_embed), jnp.float32),
            pltpu.SemaphoreType.DMA,
        ],
    )
    def gather_k(table_hbm, idx_hbm, out_hbm, idx_v, rows_v, sem):
        wid = lax.axis_index("s") * _NUM_CORES + lax.axis_index("c")
        pltpu.sync_copy(idx_hbm.at[wid], idx_v)
        base0 = wid * per_worker

        def body(j, carry):
            pltpu.async_copy(table_hbm.at[idx_v.at[j]], rows_v, sem).wait()
            pltpu.sync_copy(rows_v, out_hbm.at[pl.ds(base0 + j * chunk, chunk)])
            return carry

        lax.fori_loop(0, n_chunks, body, 0)

    return gather_k(table, idx3)


def _rtne_bf16(v):
    u = jax.lax.bitcast_convert_type(v, jnp.uint32)
    r = u + jnp.uint32(0x7FFF) + ((u >> 16) & jnp.uint32(1))
    return jax.lax.bitcast_convert_type(r & jnp.uint32(0xFFFF0000), jnp.float32)


def _tc_dense(x2, y2, at2, dt2, gath, wsp, prm, n_tokens, d_embed, blk):
    d_out = 4 * d_embed

    def body(x_ref, y_ref, at_ref, dt_ref, g_ref, w_ref, p_ref, out_ref):
        prm_v = p_ref[...]
        dx = _rtne_bf16(prm_v[0:1, :])
        dy = _rtne_bf16(prm_v[1:2, :])
        scl = prm_v[2:3, :]
        sel = prm_v[3:4, :]
        wf = prm_v[4:5, :]
        bf = prm_v[5:6, :]
        bsr = prm_v[6:7, :]
        xcol = _rtne_bf16(x_ref[...])
        ycol = _rtne_bf16(y_ref[...])
        z = (xcol * dx + ycol * dy) / scl
        feats = jnp.where(sel > 0.5, jnp.sin(z), jnp.cos(z))
        loc = jnp.dot(feats.astype(jnp.bfloat16),
                      w_ref[...].astype(jnp.bfloat16),
                      preferred_element_type=jnp.float32) + bsr
        lane = lax.broadcasted_iota(jnp.int32, (blk, d_embed), 1)
        ta = at_ref[...] * wf + bf
        td = dt_ref[...] * wf + bf
        arr = jnp.where(lane == 0, ta, jnp.sin(ta))
        dep = jnp.where(lane == 0, td, jnp.sin(td))
        out_ref[:, 0:d_embed] = loc
        out_ref[:, d_embed:2 * d_embed] = arr
        out_ref[:, 2 * d_embed:3 * d_embed] = dep
        out_ref[:, 3 * d_embed:] = g_ref[...]

    grid = n_tokens // blk
    col = pl.BlockSpec((blk, 1), lambda i: (i, 0))
    return pl.pallas_call(
        body,
        grid=(grid,),
        in_specs=[
            col, col, col, col,
            pl.BlockSpec((blk, d_embed), lambda i: (i, 0)),
            pl.BlockSpec((d_embed, d_embed), lambda i: (0, 0)),
            pl.BlockSpec((8, d_embed), lambda i: (0, 0)),
        ],
        out_specs=pl.BlockSpec((blk, d_out), lambda i: (i, 0)),
        out_shape=jax.ShapeDtypeStruct((n_tokens, d_out), jnp.float32),
    )(x2, y2, at2, dt2, gath, wsp, prm)


def kernel(region_id, x, y, arrival_time, departure_time, region_table,
           Ws, bs, w0, b0, Wt, Bt):
    b, s = region_id.shape
    n = b * s
    feat_dim, d_embed = Ws.shape

    dx, dy, scl, sel = _space2vec_consts(feat_dim, d_embed)
    wfull = jnp.concatenate([w0[None], Wt])
    bfull = jnp.concatenate([b0[None], Bt])
    prm = jnp.stack([
        jnp.asarray(dx), jnp.asarray(dy), jnp.asarray(scl), jnp.asarray(sel),
        wfull, bfull, bs,
        jnp.zeros((d_embed,), jnp.float32),
    ])
    wsp = jnp.zeros((d_embed, d_embed), jnp.float32).at[:feat_dim].set(Ws)

    gath = _sc_gather(region_table, region_id.reshape(-1).astype(jnp.int32),
                      n, d_embed)

    out = _tc_dense(
        x.reshape(n, 1), y.reshape(n, 1),
        arrival_time.reshape(n, 1), departure_time.reshape(n, 1),
        gath, wsp, prm, n, d_embed, blk=512)
    return out.reshape(b, s, 4 * d_embed)


# speedup vs baseline: 4.0965x; 1.8075x over previous
import functools

import numpy as np
import jax
import jax.numpy as jnp
from jax import lax
from jax.experimental import pallas as pl
from jax.experimental.pallas import tpu as pltpu
from jax.experimental.pallas import tpu_sc as plsc

_LAMBDA_MIN = 0.01
_LAMBDA_MAX = 10.0

_NUM_CORES = 2
_NUM_SUBCORES = 16
_NUM_WORKERS = _NUM_CORES * _NUM_SUBCORES


def _space2vec_consts(feat_dim: int, d_embed: int):
    num_scales = feat_dim // 6
    scales = _LAMBDA_MIN * (_LAMBDA_MAX / _LAMBDA_MIN) ** (
        jnp.arange(num_scales, dtype=jnp.float32) / max(num_scales - 1, 1))
    angles = jnp.array([0.0, 2.0 * np.pi / 3.0, 4.0 * np.pi / 3.0],
                       dtype=jnp.float32)
    dirx = jnp.cos(angles)
    diry = jnp.sin(angles)
    lanes = np.arange(feat_dim)
    dir_idx = np.asarray(lanes % 6 % 3)
    scale_idx = np.asarray(lanes // 6)
    pad = d_embed - feat_dim
    dx = jnp.concatenate([_rtne_bf16(dirx)[dir_idx],
                          jnp.zeros((pad,), jnp.float32)])
    dy = jnp.concatenate([_rtne_bf16(diry)[dir_idx],
                          jnp.zeros((pad,), jnp.float32)])
    scl = jnp.concatenate([scales[scale_idx], jnp.ones((pad,), jnp.float32)])
    ph = jnp.asarray(
        np.concatenate([np.where(lanes % 6 < 3, 0.0, np.pi / 2.0)
                        .astype(np.float32),
                        np.zeros((pad,), np.float32)]))
    return dx / scl, dy / scl, ph


def _sc_gather(table, idx_flat, n_tokens, d_embed):
    chunk = 128
    per_worker = n_tokens // _NUM_WORKERS
    n_chunks = per_worker // chunk
    idx3 = idx_flat.reshape(_NUM_WORKERS, n_chunks, chunk)
    mesh = plsc.VectorSubcoreMesh(core_axis_name="c", subcore_axis_name="s")

    @functools.partial(
        pl.kernel,
        out_type=jax.ShapeDtypeStruct((n_tokens, d_embed), jnp.float32),
        mesh=mesh,
        scratch_types=[
            pltpu.VMEM((n_chunks, chunk), jnp.int32),
            pltpu.VMEM((chunk, d_embed), jnp.float32),
            pltpu.SemaphoreType.DMA,
        ],
    )
    def gather_k(table_hbm, idx_hbm, out_hbm, idx_v, rows_v, sem):
        wid = lax.axis_index("s") * _NUM_CORES + lax.axis_index("c")
        pltpu.sync_copy(idx_hbm.at[wid], idx_v)
        base0 = wid * per_worker

        def body(j, carry):
            pltpu.async_copy(table_hbm.at[idx_v.at[j]], rows_v, sem).wait()
            pltpu.sync_copy(rows_v, out_hbm.at[pl.ds(base0 + j * chunk, chunk)])
            return carry

        lax.fori_loop(0, n_chunks, body, 0)

    return gather_k(table, idx3)


def _rtne_bf16(v):
    u = jax.lax.bitcast_convert_type(v, jnp.uint32)
    r = u + jnp.uint32(0x7FFF) + ((u >> 16) & jnp.uint32(1))
    return jax.lax.bitcast_convert_type(r & jnp.uint32(0xFFFF0000), jnp.float32)


_INV2PI = np.float32(0.15915494)
_T1 = np.float32(6.28125)
_T2 = np.float32(0.0019353072)
_S = [np.float32(v) for v in (0.9999997, -0.16666578, 0.008332557,
                              -0.00019812556, 2.704028e-06, -2.053326e-08)]


def _fast_sin(z):
    kf = jnp.round(z * _INV2PI)
    r = (z - kf * _T1) - kf * _T2
    r2 = r * r
    p = _S[5]
    for c in (_S[4], _S[3], _S[2], _S[1], _S[0]):
        p = p * r2 + c
    return r * p


def _tc_dense(x2, y2, at2, dt2, gath, wsp, prm, n_tokens, d_embed, blk):
    d_out = 4 * d_embed

    def body(x_ref, y_ref, at_ref, dt_ref, g_ref, w_ref, p_ref, out_ref):
        prm_v = p_ref[...]
        cx = prm_v[0:1, :]
        cy = prm_v[1:2, :]
        ph = prm_v[2:3, :]
        wf = prm_v[3:4, :]
        bf = prm_v[4:5, :]
        bsr = prm_v[5:6, :]
        xcol = _rtne_bf16(x_ref[...])
        ycol = _rtne_bf16(y_ref[...])
        z = xcol * cx + ycol * cy + ph
        feats = _fast_sin(z)
        loc = jnp.dot(feats.astype(jnp.bfloat16),
                      w_ref[...].astype(jnp.bfloat16),
                      preferred_element_type=jnp.float32) + bsr
        lane = lax.broadcasted_iota(jnp.int32, (blk, d_embed), 1)
        ta = at_ref[...] * wf + bf
        td = dt_ref[...] * wf + bf
        arr = jnp.where(lane == 0, ta, _fast_sin(ta))
        dep = jnp.where(lane == 0, td, _fast_sin(td))
        out_ref[:, 0:d_embed] = loc
        out_ref[:, d_embed:2 * d_embed] = arr
        out_ref[:, 2 * d_embed:3 * d_embed] = dep
        out_ref[:, 3 * d_embed:] = g_ref[...]

    grid = n_tokens // blk
    col = pl.BlockSpec((blk, 1), lambda i: (i, 0))
    return pl.pallas_call(
        body,
        grid=(grid,),
        in_specs=[
            col, col, col, col,
            pl.BlockSpec((blk, d_embed), lambda i: (i, 0)),
            pl.BlockSpec((d_embed, d_embed), lambda i: (0, 0)),
            pl.BlockSpec((8, d_embed), lambda i: (0, 0)),
        ],
        out_specs=pl.BlockSpec((blk, d_out), lambda i: (i, 0)),
        out_shape=jax.ShapeDtypeStruct((n_tokens, d_out), jnp.float32),
    )(x2, y2, at2, dt2, gath, wsp, prm)


def kernel(region_id, x, y, arrival_time, departure_time, region_table,
           Ws, bs, w0, b0, Wt, Bt):
    b, s = region_id.shape
    n = b * s
    feat_dim, d_embed = Ws.shape

    cx, cy, ph = _space2vec_consts(feat_dim, d_embed)
    wfull = jnp.concatenate([w0[None], Wt])
    bfull = jnp.concatenate([b0[None], Bt])
    zrow = jnp.zeros((d_embed,), jnp.float32)
    prm = jnp.stack([cx, cy, ph, wfull, bfull, bs, zrow, zrow])
    wsp = jnp.zeros((d_embed, d_embed), jnp.float32).at[:feat_dim].set(Ws)

    gath = _sc_gather(region_table, region_id.reshape(-1).astype(jnp.int32),
                      n, d_embed)

    out = _tc_dense(
        x.reshape(n, 1), y.reshape(n, 1),
        arrival_time.reshape(n, 1), departure_time.reshape(n, 1),
        gath, wsp, prm, n, d_embed, blk=512)
    return out.reshape(b, s, 4 * d_embed)


# speedup vs baseline: 4.1875x; 1.0222x over previous
import functools

import numpy as np
import jax
import jax.numpy as jnp
from jax import lax
from jax.experimental import pallas as pl
from jax.experimental.pallas import tpu as pltpu
from jax.experimental.pallas import tpu_sc as plsc

_LAMBDA_MIN = 0.01
_LAMBDA_MAX = 10.0

_NUM_CORES = 2
_NUM_SUBCORES = 16
_NUM_WORKERS = _NUM_CORES * _NUM_SUBCORES


def _space2vec_consts(feat_dim: int, d_embed: int):
    num_scales = feat_dim // 6
    scales = _LAMBDA_MIN * (_LAMBDA_MAX / _LAMBDA_MIN) ** (
        jnp.arange(num_scales, dtype=jnp.float32) / max(num_scales - 1, 1))
    angles = jnp.array([0.0, 2.0 * np.pi / 3.0, 4.0 * np.pi / 3.0],
                       dtype=jnp.float32)
    dirx = jnp.cos(angles)
    diry = jnp.sin(angles)
    lanes = np.arange(feat_dim)
    dir_idx = np.asarray(lanes % 6 % 3)
    scale_idx = np.asarray(lanes // 6)
    pad = d_embed - feat_dim
    dx = jnp.concatenate([_rtne_bf16(dirx)[dir_idx],
                          jnp.zeros((pad,), jnp.float32)])
    dy = jnp.concatenate([_rtne_bf16(diry)[dir_idx],
                          jnp.zeros((pad,), jnp.float32)])
    scl = jnp.concatenate([scales[scale_idx], jnp.ones((pad,), jnp.float32)])
    ph = jnp.asarray(
        np.concatenate([np.where(lanes % 6 < 3, 0.0, np.pi / 2.0)
                        .astype(np.float32),
                        np.zeros((pad,), np.float32)]))
    return dx / scl, dy / scl, ph


def _sc_gather(table, idx_flat, n_tokens, d_embed):
    chunk = 128
    grp = 2 * chunk
    per_worker = n_tokens // _NUM_WORKERS
    n_chunks = per_worker // chunk
    n_groups = n_chunks // 2
    idx3 = idx_flat.reshape(_NUM_WORKERS, n_chunks, chunk)
    mesh = plsc.VectorSubcoreMesh(core_axis_name="c", subcore_axis_name="s")

    @functools.partial(
        pl.kernel,
        out_type=jax.ShapeDtypeStruct((n_tokens, d_embed), jnp.float32),
        mesh=mesh,
        scratch_types=[
            pltpu.VMEM((n_chunks, chunk), jnp.int32),
            pltpu.VMEM((grp, d_embed), jnp.float32),
            pltpu.VMEM((grp, d_embed), jnp.float32),
            pltpu.SemaphoreType.DMA,
            pltpu.SemaphoreType.DMA,
        ],
    )
    def gather_k(table_hbm, idx_hbm, out_hbm, idx_v, buf_a, buf_b,
                 sem_a, sem_b):
        wid = lax.axis_index("s") * _NUM_CORES + lax.axis_index("c")
        pltpu.sync_copy(idx_hbm.at[wid], idx_v)
        base0 = wid * per_worker

        def fire(g, buf, sem):
            pltpu.async_copy(table_hbm.at[idx_v.at[2 * g]],
                             buf.at[pl.ds(0, chunk)], sem)
            pltpu.async_copy(table_hbm.at[idx_v.at[2 * g + 1]],
                             buf.at[pl.ds(chunk, chunk)], sem)

        def drain(buf, sem):
            pltpu.make_async_copy(out_hbm.at[pl.ds(0, grp)], buf, sem).wait()

        def flush(g, buf):
            pltpu.sync_copy(buf, out_hbm.at[pl.ds(base0 + g * grp, grp)])

        fire(0, buf_a, sem_a)

        def body(i, carry):
            ga = 2 * i
            fire(ga + 1, buf_b, sem_b)
            drain(buf_a, sem_a)
            flush(ga, buf_a)
            fire(ga + 2, buf_a, sem_a)
            drain(buf_b, sem_b)
            flush(ga + 1, buf_b)
            return carry

        lax.fori_loop(0, (n_groups - 1) // 2, body, 0)
        drain(buf_a, sem_a)
        flush(n_groups - 1, buf_a)

    return gather_k(table, idx3)


def _rtne_bf16(v):
    u = jax.lax.bitcast_convert_type(v, jnp.uint32)
    r = u + jnp.uint32(0x7FFF) + ((u >> 16) & jnp.uint32(1))
    return jax.lax.bitcast_convert_type(r & jnp.uint32(0xFFFF0000), jnp.float32)


_INV2PI = np.float32(0.15915494)
_TWOPI = np.float32(6.2831855)
_S = [np.float32(v) for v in (0.99998456, -0.16663256, 0.008312374,
                              -0.00019316036, 2.173132e-06)]


def _fast_sin(z):
    kf = jnp.round(z * _INV2PI)
    r = z - kf * _TWOPI
    r2 = r * r
    p = _S[4]
    for c in (_S[3], _S[2], _S[1], _S[0]):
        p = p * r2 + c
    return r * p


def _tc_dense(x2, y2, at2, dt2, gath, wsp, prm, n_tokens, d_embed, blk):
    d_out = 4 * d_embed

    def body(x_ref, y_ref, at_ref, dt_ref, g_ref, w_ref, p_ref, out_ref):
        prm_v = p_ref[...]
        cx = prm_v[0:1, :]
        cy = prm_v[1:2, :]
        ph = prm_v[2:3, :]
        wf = prm_v[3:4, :]
        bf = prm_v[4:5, :]
        bsr = prm_v[5:6, :]
        xcol = _rtne_bf16(x_ref[...])
        ycol = _rtne_bf16(y_ref[...])
        z = xcol * cx + ycol * cy + ph
        feats = _fast_sin(z)
        loc = jnp.dot(feats.astype(jnp.bfloat16),
                      w_ref[...].astype(jnp.bfloat16),
                      preferred_element_type=jnp.float32) + bsr
        lane = lax.broadcasted_iota(jnp.int32, (blk, d_embed), 1)
        ta = at_ref[...] * wf + bf
        td = dt_ref[...] * wf + bf
        arr = jnp.where(lane == 0, ta, _fast_sin(ta))
        dep = jnp.where(lane == 0, td, _fast_sin(td))
        out_ref[:, 0:d_embed] = loc
        out_ref[:, d_embed:2 * d_embed] = arr
        out_ref[:, 2 * d_embed:3 * d_embed] = dep
        out_ref[:, 3 * d_embed:] = g_ref[...]

    grid = n_tokens // blk
    col = pl.BlockSpec((blk, 1), lambda i: (i, 0))
    return pl.pallas_call(
        body,
        grid=(grid,),
        in_specs=[
            col, col, col, col,
            pl.BlockSpec((blk, d_embed), lambda i: (i, 0)),
            pl.BlockSpec((d_embed, d_embed), lambda i: (0, 0)),
            pl.BlockSpec((8, d_embed), lambda i: (0, 0)),
        ],
        out_specs=pl.BlockSpec((blk, d_out), lambda i: (i, 0)),
        out_shape=jax.ShapeDtypeStruct((n_tokens, d_out), jnp.float32),
    )(x2, y2, at2, dt2, gath, wsp, prm)


def kernel(region_id, x, y, arrival_time, departure_time, region_table,
           Ws, bs, w0, b0, Wt, Bt):
    b, s = region_id.shape
    n = b * s
    feat_dim, d_embed = Ws.shape

    cx, cy, ph = _space2vec_consts(feat_dim, d_embed)
    wfull = jnp.concatenate([w0[None], Wt])
    bfull = jnp.concatenate([b0[None], Bt])
    zrow = jnp.zeros((d_embed,), jnp.float32)
    prm = jnp.stack([cx, cy, ph, wfull, bfull, bs, zrow, zrow])
    wsp = jnp.zeros((d_embed, d_embed), jnp.float32).at[:feat_dim].set(Ws)

    gath = _sc_gather(region_table, region_id.reshape(-1).astype(jnp.int32),
                      n, d_embed)

    out = _tc_dense(
        x.reshape(n, 1), y.reshape(n, 1),
        arrival_time.reshape(n, 1), departure_time.reshape(n, 1),
        gath, wsp, prm, n, d_embed, blk=512)
    return out.reshape(b, s, 4 * d_embed)


# speedup vs baseline: 5.3512x; 1.2779x over previous
import functools

import numpy as np
import jax
import jax.numpy as jnp
from jax import lax
from jax.experimental import pallas as pl
from jax.experimental.pallas import tpu as pltpu
from jax.experimental.pallas import tpu_sc as plsc

_LAMBDA_MIN = 0.01
_LAMBDA_MAX = 10.0

_NUM_CORES = 2
_NUM_SUBCORES = 16
_NUM_WORKERS = _NUM_CORES * _NUM_SUBCORES


def _space2vec_consts(feat_dim: int, d_embed: int):
    num_scales = feat_dim // 6
    scales = _LAMBDA_MIN * (_LAMBDA_MAX / _LAMBDA_MIN) ** (
        jnp.arange(num_scales, dtype=jnp.float32) / max(num_scales - 1, 1))
    angles = jnp.array([0.0, 2.0 * np.pi / 3.0, 4.0 * np.pi / 3.0],
                       dtype=jnp.float32)
    dirx = jnp.cos(angles)
    diry = jnp.sin(angles)
    lanes = np.arange(feat_dim)
    dir_idx = np.asarray(lanes % 6 % 3)
    scale_idx = np.asarray(lanes // 6)
    pad = d_embed - feat_dim
    dx = jnp.concatenate([_rtne_bf16(dirx)[dir_idx],
                          jnp.zeros((pad,), jnp.float32)])
    dy = jnp.concatenate([_rtne_bf16(diry)[dir_idx],
                          jnp.zeros((pad,), jnp.float32)])
    scl = jnp.concatenate([scales[scale_idx], jnp.ones((pad,), jnp.float32)])
    ph = jnp.asarray(
        np.concatenate([np.where(lanes % 6 < 3, 0.0, np.pi / 2.0)
                        .astype(np.float32),
                        np.zeros((pad,), np.float32)]))
    return dx / scl, dy / scl, ph


def _sc_gather(table, idx_flat, n_tokens, d_embed):
    chunk = 128
    grp = 2 * chunk
    per_worker = n_tokens // _NUM_WORKERS
    n_chunks = per_worker // chunk
    n_groups = n_chunks // 2
    idx3 = idx_flat.reshape(_NUM_WORKERS, n_chunks, chunk)
    mesh = plsc.VectorSubcoreMesh(core_axis_name="c", subcore_axis_name="s")

    @functools.partial(
        pl.kernel,
        out_type=jax.ShapeDtypeStruct((n_tokens, d_embed), jnp.float32),
        mesh=mesh,
        scratch_types=[
            pltpu.VMEM((n_chunks, chunk), jnp.int32),
            pltpu.VMEM((grp, d_embed), jnp.float32),
            pltpu.VMEM((grp, d_embed), jnp.float32),
            pltpu.SemaphoreType.DMA,
            pltpu.SemaphoreType.DMA,
        ],
    )
    def gather_k(table_hbm, idx_hbm, out_hbm, idx_v, buf_a, buf_b,
                 sem_a, sem_b):
        wid = lax.axis_index("s") * _NUM_CORES + lax.axis_index("c")
        pltpu.sync_copy(idx_hbm.at[wid], idx_v)
        base0 = wid * per_worker

        def fire(g, buf, sem):
            pltpu.async_copy(table_hbm.at[idx_v.at[2 * g]],
                             buf.at[pl.ds(0, chunk)], sem)
            pltpu.async_copy(table_hbm.at[idx_v.at[2 * g + 1]],
                             buf.at[pl.ds(chunk, chunk)], sem)

        def drain(buf, sem):
            pltpu.make_async_copy(out_hbm.at[pl.ds(0, grp)], buf, sem).wait()

        def flush(g, buf):
            pltpu.sync_copy(buf, out_hbm.at[pl.ds(base0 + g * grp, grp)])

        fire(0, buf_a, sem_a)

        def body(i, carry):
            ga = 2 * i
            fire(ga + 1, buf_b, sem_b)
            drain(buf_a, sem_a)
            flush(ga, buf_a)
            fire(ga + 2, buf_a, sem_a)
            drain(buf_b, sem_b)
            flush(ga + 1, buf_b)
            return carry

        lax.fori_loop(0, (n_groups - 1) // 2, body, 0)
        drain(buf_a, sem_a)
        flush(n_groups - 1, buf_a)

    return gather_k(table, idx3)


def _rtne_bf16(v):
    u = jax.lax.bitcast_convert_type(v, jnp.uint32)
    r = u + jnp.uint32(0x7FFF) + ((u >> 16) & jnp.uint32(1))
    return jax.lax.bitcast_convert_type(r & jnp.uint32(0xFFFF0000), jnp.float32)


_INV2PI = np.float32(0.15915494)
_TWOPI = np.float32(6.2831855)
_S = [np.float32(v) for v in (0.99998456, -0.16663256, 0.008312374,
                              -0.00019316036, 2.173132e-06)]


def _fast_sin(z):
    kf = jnp.round(z * _INV2PI)
    r = z - kf * _TWOPI
    r2 = r * r
    p = _S[4]
    for c in (_S[3], _S[2], _S[1], _S[0]):
        p = p * r2 + c
    return r * p


def _tc_dense(x2, y2, at2, dt2, gath, wsp, prm, n_tokens, d_embed, blk):
    d_out = 4 * d_embed

    def body(x_ref, y_ref, at_ref, dt_ref, g_ref, w_ref, p_ref, out_ref):
        prm_v = p_ref[...]
        cx = prm_v[0:1, :]
        cy = prm_v[1:2, :]
        ph = prm_v[2:3, :]
        wf = prm_v[3:4, :]
        bf = prm_v[4:5, :]
        bsr = prm_v[5:6, :]
        xcol = _rtne_bf16(x_ref[...])
        ycol = _rtne_bf16(y_ref[...])
        z = xcol * cx + ycol * cy + ph
        feats = _fast_sin(z)
        loc = jnp.dot(feats.astype(jnp.bfloat16),
                      w_ref[...].astype(jnp.bfloat16),
                      preferred_element_type=jnp.float32) + bsr
        lane = lax.broadcasted_iota(jnp.int32, (blk, d_embed), 1)
        ta = at_ref[...] * wf + bf
        td = dt_ref[...] * wf + bf
        arr = jnp.where(lane == 0, ta, _fast_sin(ta))
        dep = jnp.where(lane == 0, td, _fast_sin(td))
        out_ref[:, 0:d_embed] = loc
        out_ref[:, d_embed:2 * d_embed] = arr
        out_ref[:, 2 * d_embed:3 * d_embed] = dep
        out_ref[:, 3 * d_embed:] = g_ref[...]

    grid = n_tokens // blk
    col = pl.BlockSpec((blk, 1), lambda i: (i, 0))
    return pl.pallas_call(
        body,
        grid=(grid,),
        in_specs=[
            col, col, col, col,
            pl.BlockSpec((blk, d_embed), lambda i: (i, 0)),
            pl.BlockSpec((d_embed, d_embed), lambda i: (0, 0)),
            pl.BlockSpec((8, d_embed), lambda i: (0, 0)),
        ],
        out_specs=pl.BlockSpec((blk, d_out), lambda i: (i, 0)),
        out_shape=jax.ShapeDtypeStruct((n_tokens, d_out), jnp.float32),
    )(x2, y2, at2, dt2, gath, wsp, prm)


def kernel(region_id, x, y, arrival_time, departure_time, region_table,
           Ws, bs, w0, b0, Wt, Bt):
    b, s = region_id.shape
    n = b * s
    feat_dim, d_embed = Ws.shape

    cx, cy, ph = _space2vec_consts(feat_dim, d_embed)
    wfull = jnp.concatenate([w0[None], Wt])
    bfull = jnp.concatenate([b0[None], Bt])
    zrow = jnp.zeros((d_embed,), jnp.float32)
    prm = jnp.stack([cx, cy, ph, wfull, bfull, bs, zrow, zrow])
    wsp = jnp.zeros((d_embed, d_embed), jnp.float32).at[:feat_dim].set(Ws)

    gath = _sc_gather(region_table, region_id.reshape(-1).astype(jnp.int32),
                      n, d_embed)

    out = _tc_dense(
        x.reshape(n, 1), y.reshape(n, 1),
        arrival_time.reshape(n, 1), departure_time.reshape(n, 1),
        gath, wsp, prm, n, d_embed, blk=2048)
    return out.reshape(b, s, 4 * d_embed)


# speedup vs baseline: 5.4967x; 1.0272x over previous
import functools

import numpy as np
import jax
import jax.numpy as jnp
from jax import lax
from jax.experimental import pallas as pl
from jax.experimental.pallas import tpu as pltpu
from jax.experimental.pallas import tpu_sc as plsc

_LAMBDA_MIN = 0.01
_LAMBDA_MAX = 10.0

_NUM_CORES = 2
_NUM_SUBCORES = 16
_NUM_WORKERS = _NUM_CORES * _NUM_SUBCORES


def _space2vec_consts(feat_dim: int, d_embed: int):
    num_scales = feat_dim // 6
    scales = _LAMBDA_MIN * (_LAMBDA_MAX / _LAMBDA_MIN) ** (
        jnp.arange(num_scales, dtype=jnp.float32) / max(num_scales - 1, 1))
    angles = jnp.array([0.0, 2.0 * np.pi / 3.0, 4.0 * np.pi / 3.0],
                       dtype=jnp.float32)
    dirx = jnp.cos(angles)
    diry = jnp.sin(angles)
    lanes = np.arange(feat_dim)
    dir_idx = np.asarray(lanes % 6 % 3)
    scale_idx = np.asarray(lanes // 6)
    pad = d_embed - feat_dim
    dx = jnp.concatenate([_rtne_bf16(dirx)[dir_idx],
                          jnp.zeros((pad,), jnp.float32)])
    dy = jnp.concatenate([_rtne_bf16(diry)[dir_idx],
                          jnp.zeros((pad,), jnp.float32)])
    scl = jnp.concatenate([scales[scale_idx], jnp.ones((pad,), jnp.float32)])
    ph = jnp.asarray(
        np.concatenate([np.where(lanes % 6 < 3, 0.0, np.pi / 2.0)
                        .astype(np.float32),
                        np.zeros((pad,), np.float32)]))
    return dx / scl, dy / scl, ph


def _sc_gather(table, idx_flat, n_tokens, d_embed):
    chunk = 128
    grp = 2 * chunk
    per_worker = n_tokens // _NUM_WORKERS
    n_chunks = per_worker // chunk
    n_groups = n_chunks // 2
    idx3 = idx_flat.reshape(_NUM_WORKERS, n_chunks, chunk)
    mesh = plsc.VectorSubcoreMesh(core_axis_name="c", subcore_axis_name="s")

    @functools.partial(
        pl.kernel,
        out_type=jax.ShapeDtypeStruct((n_tokens, d_embed), jnp.float32),
        mesh=mesh,
        scratch_types=[
            pltpu.VMEM((n_chunks, chunk), jnp.int32),
            pltpu.VMEM((grp, d_embed), jnp.float32),
            pltpu.VMEM((grp, d_embed), jnp.float32),
            pltpu.SemaphoreType.DMA,
            pltpu.SemaphoreType.DMA,
        ],
    )
    def gather_k(table_hbm, idx_hbm, out_hbm, idx_v, buf_a, buf_b,
                 sem_a, sem_b):
        wid = lax.axis_index("s") * _NUM_CORES + lax.axis_index("c")
        pltpu.sync_copy(idx_hbm.at[wid], idx_v)
        base0 = wid * per_worker

        def fire(g, buf, sem):
            pltpu.async_copy(table_hbm.at[idx_v.at[2 * g]],
                             buf.at[pl.ds(0, chunk)], sem)
            pltpu.async_copy(table_hbm.at[idx_v.at[2 * g + 1]],
                             buf.at[pl.ds(chunk, chunk)], sem)

        def drain(buf, sem):
            pltpu.make_async_copy(out_hbm.at[pl.ds(0, grp)], buf, sem).wait()

        def flush(g, buf):
            pltpu.sync_copy(buf, out_hbm.at[pl.ds(base0 + g * grp, grp)])

        fire(0, buf_a, sem_a)

        def body(i, carry):
            ga = 2 * i
            fire(ga + 1, buf_b, sem_b)
            drain(buf_a, sem_a)
            flush(ga, buf_a)
            fire(ga + 2, buf_a, sem_a)
            drain(buf_b, sem_b)
            flush(ga + 1, buf_b)
            return carry

        lax.fori_loop(0, (n_groups - 1) // 2, body, 0)
        drain(buf_a, sem_a)
        flush(n_groups - 1, buf_a)

    return gather_k(table, idx3)


def _rtne_bf16(v):
    u = jax.lax.bitcast_convert_type(v, jnp.uint32)
    r = u + jnp.uint32(0x7FFF) + ((u >> 16) & jnp.uint32(1))
    return jax.lax.bitcast_convert_type(r & jnp.uint32(0xFFFF0000), jnp.float32)


_INV2PI = np.float32(0.15915494)
_TWOPI = np.float32(6.2831855)
_S = [np.float32(v) for v in (0.99998456, -0.16663256, 0.008312374,
                              -0.00019316036, 2.173132e-06)]


def _fast_sin(z):
    kf = jnp.round(z * _INV2PI)
    r = z - kf * _TWOPI
    r2 = r * r
    p = _S[4]
    for c in (_S[3], _S[2], _S[1], _S[0]):
        p = p * r2 + c
    return r * p


def _tc_dense(x2, y2, at2, dt2, gath, wsp, prm, n_tokens, d_embed, blk):
    d_out = 4 * d_embed

    def body(x_ref, y_ref, at_ref, dt_ref, g_ref, w_ref, p_ref, out_ref):
        prm_v = p_ref[...]
        cx = prm_v[0:1, :]
        cy = prm_v[1:2, :]
        ph = prm_v[2:3, :]
        wf = prm_v[3:4, :]
        bf = prm_v[4:5, :]
        bsr = prm_v[5:6, :]
        xcol = _rtne_bf16(x_ref[...])
        ycol = _rtne_bf16(y_ref[...])
        z = xcol * cx + ycol * cy + ph
        feats = _fast_sin(z)
        loc = jnp.dot(feats.astype(jnp.bfloat16),
                      w_ref[...].astype(jnp.bfloat16),
                      preferred_element_type=jnp.float32) + bsr
        lane = lax.broadcasted_iota(jnp.int32, (blk, d_embed), 1)
        ta = at_ref[...] * wf + bf
        td = dt_ref[...] * wf + bf
        arr = jnp.where(lane == 0, ta, _fast_sin(ta))
        dep = jnp.where(lane == 0, td, _fast_sin(td))
        out_ref[:, 0:d_embed] = loc
        out_ref[:, d_embed:2 * d_embed] = arr
        out_ref[:, 2 * d_embed:3 * d_embed] = dep
        out_ref[:, 3 * d_embed:] = g_ref[...]

    grid = n_tokens // blk
    col = pl.BlockSpec((blk, 1), lambda i: (i, 0))
    return pl.pallas_call(
        body,
        grid=(grid,),
        in_specs=[
            col, col, col, col,
            pl.BlockSpec((blk, d_embed), lambda i: (i, 0)),
            pl.BlockSpec((d_embed, d_embed), lambda i: (0, 0)),
            pl.BlockSpec((8, d_embed), lambda i: (0, 0)),
        ],
        out_specs=pl.BlockSpec((blk, d_out), lambda i: (i, 0)),
        out_shape=jax.ShapeDtypeStruct((n_tokens, d_out), jnp.float32),
    )(x2, y2, at2, dt2, gath, wsp, prm)


def kernel(region_id, x, y, arrival_time, departure_time, region_table,
           Ws, bs, w0, b0, Wt, Bt):
    b, s = region_id.shape
    n = b * s
    feat_dim, d_embed = Ws.shape

    cx, cy, ph = _space2vec_consts(feat_dim, d_embed)
    wfull = jnp.concatenate([w0[None], Wt])
    bfull = jnp.concatenate([b0[None], Bt])
    zrow = jnp.zeros((d_embed,), jnp.float32)
    prm = jnp.stack([cx, cy, ph, wfull, bfull, bs, zrow, zrow])
    wsp = jnp.zeros((d_embed, d_embed), jnp.float32).at[:feat_dim].set(Ws)

    gath = _sc_gather(region_table, region_id.reshape(-1).astype(jnp.int32),
                      n, d_embed)

    out = _tc_dense(
        x.reshape(n, 1), y.reshape(n, 1),
        arrival_time.reshape(n, 1), departure_time.reshape(n, 1),
        gath, wsp, prm, n, d_embed, blk=4096)
    return out.reshape(b, s, 4 * d_embed)
